# Initial kernel scaffold; baseline (speedup 1.0000x reference)
import jax
import jax.numpy as jnp
from jax.experimental import pallas as pl


def kernel(frag_node, frag_edge, edge_index, graph_ids, W_init, W_edge, w_att, W_msg, Wz_a, Uz_a, Wr_a, Ur_a, Wn_a, Un_a, w_att_m, W_msg_m, Wz_m, Uz_m, Wr_m, Ur_m, Wn_m, Un_m):
    raise NotImplementedError("write your pallas kernel here")


# speedup vs baseline: 1.4681x; 1.4681x over previous
import jax, jax.numpy as jnp
from jax.experimental import pallas as pl

L = 2
T = 2

def _gru(x, h, Wz, Uz, Wr, Ur, Wn, Un):
    z = jax.nn.sigmoid(x @ Wz + h @ Uz)
    r = jax.nn.sigmoid(x @ Wr + h @ Ur)
    n = jnp.tanh(x @ Wn + r * (h @ Un))
    return (1.0 - z) * n + z * h

def kernel(frag_node, frag_edge, edge_index, graph_ids, W_init, W_edge, w_att, W_msg,
             Wz_a, Uz_a, Wr_a, Ur_a, Wn_a, Un_a,
             w_att_m, W_msg_m, Wz_m, Uz_m, Wr_m, Ur_m, Wn_m, Un_m):
    N, D = frag_node.shape
    G = 256
    src = edge_index[0]; dst = edge_index[1]
    h = jax.nn.leaky_relu(frag_node @ W_init)
    Eb = frag_edge @ W_edge[D:]
    We_top = W_edge[:D]
    wa_top = w_att[:D, 0]; wa_bot = w_att[D:, 0]
    for _ in range(L):
        Hn = h @ We_top
        hd = h @ wa_top
        e = jax.nn.leaky_relu(Hn[src] + Eb)
        s = e @ wa_bot
        logit = jax.nn.leaky_relu(hd[dst] + s)
        ex = jnp.exp(logit)
        sig = jax.ops.segment_sum(ex, dst, num_segments=N)
        U = jax.ops.segment_sum(ex[:, None] * e, dst, num_segments=N)
        S = U / (sig + 1e-9)[:, None]
        ctx = jax.nn.elu(S @ W_msg)
        h = _gru(ctx, h, Wz_a, Uz_a, Wr_a, Ur_a, Wn_a, Un_a)
    g = jax.ops.segment_sum(h, graph_ids, num_segments=G)
    wm_top = w_att_m[:D, 0]; wm_bot = w_att_m[D:, 0]
    for _ in range(T):
        gl = g @ wm_top
        hl = h @ wm_bot
        logit = jax.nn.leaky_relu(gl[graph_ids] + hl)
        ex = jnp.exp(logit)
        sig = jax.ops.segment_sum(ex, graph_ids, num_segments=G)
        U = jax.ops.segment_sum(ex[:, None] * h, graph_ids, num_segments=G)
        S = U / (sig + 1e-9)[:, None]
        ctx = jax.nn.elu(S @ W_msg_m)
        g = _gru(ctx, g, Wz_m, Uz_m, Wr_m, Ur_m, Wn_m, Un_m)
    return g


# speedup vs baseline: 6.9187x; 4.7128x over previous
import jax
import jax.numpy as jnp
from jax import lax
from jax.experimental import pallas as pl
from jax.experimental.pallas import tpu as pltpu
from jax.experimental.pallas import tpu_sc as plsc

_N = 10000
_E = 320000
_D = 128
_DE = 16
_G = 256
_L = 2
_T = 2

_NC, _NS = 2, 16
_NW = _NC * _NS
_EW = _E // _NW
_CK = 80
_NCHUNK = _EW // _CK
_SPAN = 640
_SPAN_LAST = _N - (_NS - 1) * _SPAN
_NP = _NS * _SPAN


def _edge_body(src_h, dst_h, hn_h, hd_h, eb_h, w_h,
               u_out, sig_out,
               src_v, dst_v, rows_v, eb_v, exe_v, exc_v, sigbuf_v,
               hd_l, w_l, u_sh, sig_sh, gsem):
    c = lax.axis_index("c")
    s = lax.axis_index("s")
    wid = c * _NS + s
    z16f = jnp.zeros((16,), jnp.float32)

    @pl.loop(0, _CK)
    def _zrow(r):
        for k in range(8):
            exe_v[r, pl.ds(k * 16, 16)] = z16f
    for i in range(_SPAN // 16):
        sigbuf_v[pl.ds(i * 16, 16)] = z16f

    @pl.when(s < _NS - 1)
    def _():
        for b in range(_SPAN // _CK):
            pltpu.sync_copy(exe_v, u_sh.at[pl.ds(s * _SPAN + b * _CK, _CK), :])

    @pl.when(s == _NS - 1)
    def _():
        for b in range(_SPAN_LAST // _CK):
            pltpu.sync_copy(exe_v, u_sh.at[pl.ds(s * _SPAN + b * _CK, _CK), :])

    pltpu.sync_copy(sigbuf_v, sig_sh.at[pl.ds(s * _SPAN, _SPAN)])

    pltpu.sync_copy(hd_h, hd_l)
    pltpu.sync_copy(w_h, w_l)
    plsc.subcore_barrier()

    wvs = [w_l[pl.ds(k * 16, 16)] for k in range(8)]
    iota16 = lax.iota(jnp.int32, 16)
    z16 = jnp.zeros((16,), jnp.int32)
    ebase = wid * _EW

    @pl.loop(0, _NCHUNK)
    def _chunk(ci):
        base = ebase + ci * _CK
        pltpu.sync_copy(src_h.at[pl.ds(base, _CK)], src_v)
        pltpu.sync_copy(dst_h.at[pl.ds(base, _CK)], dst_v)
        pltpu.async_copy(hn_h.at[src_v], rows_v, gsem).wait()
        pltpu.sync_copy(eb_h.at[pl.ds(base, _CK), :], eb_v)

        @pl.loop(0, _CK // 16)
        def _group(j):
            r0 = j * 16
            riv = r0 + iota16
            dstv = plsc.load_gather(dst_v, [riv])
            hdj = plsc.load_gather(hd_l, [dstv])
            sv = jnp.zeros((16,), jnp.float32)
            for e in range(16):
                row = r0 + e
                acc = None
                for k in range(8):
                    hv = rows_v[row, pl.ds(k * 16, 16)]
                    ev = eb_v[row, pl.ds(k * 16, 16)]
                    x = hv + ev
                    evec = jnp.where(x >= 0, x, 0.01 * x)
                    exe_v[row, pl.ds(k * 16, 16)] = evec
                    acc = evec * wvs[k] if acc is None else acc + evec * wvs[k]
                sv = jnp.where(iota16 == e, jnp.sum(acc), sv)
            lg = hdj + sv
            logit = jnp.where(lg >= 0, lg, 0.01 * lg)
            ex = jnp.exp(logit)
            plsc.store_scatter(exc_v, [riv], ex)
            for e in range(16):
                row = r0 + e
                exs = jnp.sum(jnp.where(iota16 == e, ex, 0.0))
                for k in range(8):
                    exe_v[row, pl.ds(k * 16, 16)] = (
                        exe_v[row, pl.ds(k * 16, 16)] * exs)

        pltpu.sync_copy(exe_v, u_sh.at[dst_v], add=True)
        pltpu.sync_copy(exc_v, sig_sh.at[dst_v], add=True)

    plsc.subcore_barrier()

    @pl.when(s < _NS - 1)
    def _():
        for b in range(_SPAN // _CK):
            r0 = s * _SPAN + b * _CK
            pltpu.sync_copy(u_sh.at[pl.ds(r0, _CK), :], exe_v)
            pltpu.sync_copy(exe_v, u_out.at[c, pl.ds(r0, _CK), :])

    @pl.when(s == _NS - 1)
    def _():
        for b in range(_SPAN_LAST // _CK):
            r0 = s * _SPAN + b * _CK
            pltpu.sync_copy(u_sh.at[pl.ds(r0, _CK), :], exe_v)
            pltpu.sync_copy(exe_v, u_out.at[c, pl.ds(r0, _CK), :])

    pltpu.sync_copy(sig_sh.at[pl.ds(s * _SPAN, _SPAN)], sigbuf_v)
    pltpu.sync_copy(sigbuf_v, sig_out.at[pl.ds(c * _NP + s * _SPAN, _SPAN)])


_edge_kernel = pl.kernel(
    _edge_body,
    out_type=(jax.ShapeDtypeStruct((_NC, _N, _D), jnp.float32),
              jax.ShapeDtypeStruct((_NC * _NP,), jnp.float32)),
    mesh=plsc.VectorSubcoreMesh(core_axis_name="c", subcore_axis_name="s",
                                num_cores=_NC, num_subcores=_NS),
    compiler_params=pltpu.CompilerParams(needs_layout_passes=False),
    scratch_types=[
        pltpu.VMEM((_CK,), jnp.int32),
        pltpu.VMEM((_CK,), jnp.int32),
        pltpu.VMEM((_CK, _D), jnp.float32),
        pltpu.VMEM((_CK, _D), jnp.float32),
        pltpu.VMEM((_CK, _D), jnp.float32),
        pltpu.VMEM((_CK,), jnp.float32),
        pltpu.VMEM((_SPAN,), jnp.float32),
        pltpu.VMEM((_N,), jnp.float32),
        pltpu.VMEM((_D,), jnp.float32),
        pltpu.VMEM_SHARED((_N, _D), jnp.float32),
        pltpu.VMEM_SHARED((_NP,), jnp.float32),
        pltpu.SemaphoreType.DMA,
    ],
)


def _gru(x, h, Wz, Uz, Wr, Ur, Wn, Un):
    z = jax.nn.sigmoid(x @ Wz + h @ Uz)
    r = jax.nn.sigmoid(x @ Wr + h @ Ur)
    n = jnp.tanh(x @ Wn + r * (h @ Un))
    return (1.0 - z) * n + z * h


def kernel(frag_node, frag_edge, edge_index, graph_ids, W_init, W_edge, w_att, W_msg,
           Wz_a, Uz_a, Wr_a, Ur_a, Wn_a, Un_a,
           w_att_m, W_msg_m, Wz_m, Uz_m, Wr_m, Ur_m, Wn_m, Un_m):
    src = edge_index[0]
    dst = edge_index[1]
    h = jax.nn.leaky_relu(frag_node @ W_init)
    Eb = frag_edge @ W_edge[_D:]
    We_top = W_edge[:_D]
    wa_top = w_att[:_D]
    wa_bot = w_att[_D:, 0]
    for _ in range(_L):
        Hn = h @ We_top
        hd = (h @ wa_top)[:, 0]
        U2, sigf = _edge_kernel(src, dst, Hn, hd, Eb, wa_bot)
        sig2 = sigf.reshape(_NC, _NP)[:, :_N]
        S = (U2[0] + U2[1]) / (sig2[0] + sig2[1] + 1e-9)[:, None]
        ctx = jax.nn.elu(S @ W_msg)
        h = _gru(ctx, h, Wz_a, Uz_a, Wr_a, Ur_a, Wn_a, Un_a)
    g = jax.ops.segment_sum(h, graph_ids, num_segments=_G)
    wm_top = w_att_m[:_D, 0]
    wm_bot = w_att_m[_D:, 0]
    for _ in range(_T):
        gl = g @ wm_top
        hl = h @ wm_bot
        logit = jax.nn.leaky_relu(gl[graph_ids] + hl)
        ex = jnp.exp(logit)
        sig = jax.ops.segment_sum(ex, graph_ids, num_segments=_G)
        U = jax.ops.segment_sum(ex[:, None] * h, graph_ids, num_segments=_G)
        S = U / (sig + 1e-9)[:, None]
        ctx = jax.nn.elu(S @ W_msg_m)
        g = _gru(ctx, g, Wz_m, Uz_m, Wr_m, Ur_m, Wn_m, Un_m)
    return g


# speedup vs baseline: 7.9307x; 1.1463x over previous
import jax
import jax.numpy as jnp
from jax import lax
from jax.experimental import pallas as pl
from jax.experimental.pallas import tpu as pltpu
from jax.experimental.pallas import tpu_sc as plsc

_N = 10000
_E = 320000
_D = 128
_DE = 16
_G = 256
_L = 2
_T = 2

_NC, _NS = 2, 16
_NW = _NC * _NS
_EW = _E // _NW
_CK = 80
_NCHUNK = _EW // _CK
_SPAN = 640
_SPAN_LAST = _N - (_NS - 1) * _SPAN
_NP = _NS * _SPAN


def _edge_body(src_h, dst_h, hn_h, hd_h, eb_h, w_h,
               u_out, sig_out,
               src_v, dst_v, rows_v, eb_v, exe_v, exc_v, sigbuf_v,
               hd_l, w_l, u_sh, sig_sh, gsem):
    c = lax.axis_index("c")
    s = lax.axis_index("s")
    wid = c * _NS + s
    z16f = jnp.zeros((16,), jnp.float32)

    @pl.loop(0, _CK)
    def _zrow(r):
        for k in range(8):
            exe_v[r, pl.ds(k * 16, 16)] = z16f
    for i in range(_SPAN // 16):
        sigbuf_v[pl.ds(i * 16, 16)] = z16f

    @pl.when(s < _NS - 1)
    def _():
        pltpu.sync_copy(sigbuf_v, sig_sh.at[pl.ds(s * _SPAN, _SPAN)])

    @pl.when(s == _NS - 1)
    def _():
        pltpu.sync_copy(sigbuf_v.at[pl.ds(0, _SPAN_LAST)],
                        sig_sh.at[pl.ds(s * _SPAN, _SPAN_LAST)])

    @pl.when(s < _NS - 1)
    def _():
        for b in range(_SPAN // _CK):
            pltpu.sync_copy(exe_v, u_sh.at[pl.ds(s * _SPAN + b * _CK, _CK), :])

    @pl.when(s == _NS - 1)
    def _():
        for b in range(_SPAN_LAST // _CK):
            pltpu.sync_copy(exe_v, u_sh.at[pl.ds(s * _SPAN + b * _CK, _CK), :])

    pltpu.sync_copy(hd_h, hd_l)
    pltpu.sync_copy(w_h, w_l)
    plsc.subcore_barrier()

    wvs = [w_l[pl.ds(k * 16, 16)] for k in range(8)]
    iota16 = lax.iota(jnp.int32, 16)
    z16 = jnp.zeros((16,), jnp.int32)
    ebase = wid * _EW

    @pl.loop(0, _NCHUNK)
    def _chunk(ci):
        base = ebase + ci * _CK
        pltpu.sync_copy(src_h.at[pl.ds(base, _CK)], src_v)
        pltpu.sync_copy(dst_h.at[pl.ds(base, _CK)], dst_v)
        pltpu.async_copy(hn_h.at[src_v], rows_v, gsem).wait()
        pltpu.sync_copy(eb_h.at[pl.ds(base, _CK), :], eb_v)

        @pl.loop(0, _CK // 16)
        def _group(j):
            r0 = j * 16
            riv = r0 + iota16
            dstv = plsc.load_gather(dst_v, [riv])
            hdj = plsc.load_gather(hd_l, [dstv])
            sv = jnp.zeros((16,), jnp.float32)
            for e in range(16):
                row = r0 + e
                acc = None
                for k in range(8):
                    hv = rows_v[row, pl.ds(k * 16, 16)]
                    ev = eb_v[row, pl.ds(k * 16, 16)]
                    x = hv + ev
                    evec = jnp.where(x >= 0, x, 0.01 * x)
                    exe_v[row, pl.ds(k * 16, 16)] = evec
                    acc = evec * wvs[k] if acc is None else acc + evec * wvs[k]
                sv = jnp.where(iota16 == e, jnp.sum(acc), sv)
            lg = hdj + sv
            logit = jnp.where(lg >= 0, lg, 0.01 * lg)
            ex = jnp.exp(logit)
            plsc.store_scatter(exc_v, [riv], ex)
            for e in range(16):
                row = r0 + e
                exs = jnp.sum(jnp.where(iota16 == e, ex, 0.0))
                for k in range(8):
                    exe_v[row, pl.ds(k * 16, 16)] = (
                        exe_v[row, pl.ds(k * 16, 16)] * exs)

        pltpu.sync_copy(exe_v, u_sh.at[dst_v], add=True)
        pltpu.sync_copy(exc_v, sig_sh.at[dst_v], add=True)

    plsc.subcore_barrier()

    @pl.when(s < _NS - 1)
    def _():
        for b in range(_SPAN // _CK):
            r0 = s * _SPAN + b * _CK
            pltpu.sync_copy(u_sh.at[pl.ds(r0, _CK), :], exe_v)
            pltpu.sync_copy(exe_v, u_out.at[c, pl.ds(r0, _CK), :])

    @pl.when(s == _NS - 1)
    def _():
        for b in range(_SPAN_LAST // _CK):
            r0 = s * _SPAN + b * _CK
            pltpu.sync_copy(u_sh.at[pl.ds(r0, _CK), :], exe_v)
            pltpu.sync_copy(exe_v, u_out.at[c, pl.ds(r0, _CK), :])

    @pl.when(s < _NS - 1)
    def _():
        pltpu.sync_copy(sig_sh.at[pl.ds(s * _SPAN, _SPAN)], sigbuf_v)
        pltpu.sync_copy(sigbuf_v, sig_out.at[pl.ds(c * _N + s * _SPAN, _SPAN)])

    @pl.when(s == _NS - 1)
    def _():
        pltpu.sync_copy(sig_sh.at[pl.ds(s * _SPAN, _SPAN_LAST)],
                        sigbuf_v.at[pl.ds(0, _SPAN_LAST)])
        pltpu.sync_copy(sigbuf_v.at[pl.ds(0, _SPAN_LAST)],
                        sig_out.at[pl.ds(c * _N + s * _SPAN, _SPAN_LAST)])


_edge_kernel = pl.kernel(
    _edge_body,
    out_type=(jax.ShapeDtypeStruct((_NC, _N, _D), jnp.float32),
              jax.ShapeDtypeStruct((_NC * _N,), jnp.float32)),
    mesh=plsc.VectorSubcoreMesh(core_axis_name="c", subcore_axis_name="s",
                                num_cores=_NC, num_subcores=_NS),
    compiler_params=pltpu.CompilerParams(needs_layout_passes=False),
    scratch_types=[
        pltpu.VMEM((_CK,), jnp.int32),
        pltpu.VMEM((_CK,), jnp.int32),
        pltpu.VMEM((_CK, _D), jnp.float32),
        pltpu.VMEM((_CK, _D), jnp.float32),
        pltpu.VMEM((_CK, _D), jnp.float32),
        pltpu.VMEM((_CK,), jnp.float32),
        pltpu.VMEM((_SPAN,), jnp.float32),
        pltpu.VMEM((_N,), jnp.float32),
        pltpu.VMEM((_D,), jnp.float32),
        pltpu.VMEM_SHARED((_N, _D), jnp.float32),
        pltpu.VMEM_SHARED((_N,), jnp.float32),
        pltpu.SemaphoreType.DMA,
    ],
)


def _gru(x, h, Wz, Uz, Wr, Ur, Wn, Un):
    z = jax.nn.sigmoid(x @ Wz + h @ Uz)
    r = jax.nn.sigmoid(x @ Wr + h @ Ur)
    n = jnp.tanh(x @ Wn + r * (h @ Un))
    return (1.0 - z) * n + z * h


def _lrelu(x):
    return jnp.where(x >= 0, x, 0.01 * x)


def _elu(x):
    return jnp.where(x > 0, x, jnp.exp(jnp.minimum(x, 0.0)) - 1.0)


_EBLK = 8000


def _eb_body(fe_ref, w_ref, out_ref):
    out_ref[...] = jnp.dot(fe_ref[...], w_ref[...],
                           preferred_element_type=jnp.float32)


def _compute_eb(frag_edge, w_bot):
    return pl.pallas_call(
        _eb_body,
        grid=(_E // _EBLK,),
        in_specs=[pl.BlockSpec((_EBLK, _DE), lambda i: (i, 0)),
                  pl.BlockSpec((_DE, _D), lambda i: (0, 0))],
        out_specs=pl.BlockSpec((_EBLK, _D), lambda i: (i, 0)),
        out_shape=jax.ShapeDtypeStruct((_E, _D), jnp.float32),
    )(frag_edge, w_bot)


_NBLK = 1000


def _init_body(x_ref, wi_ref, we_ref, wa_ref, h_ref, hn_ref, hd_ref):
    h = _lrelu(jnp.dot(x_ref[...], wi_ref[...],
                       preferred_element_type=jnp.float32))
    h_ref[...] = h
    hn_ref[...] = jnp.dot(h, we_ref[...], preferred_element_type=jnp.float32)
    hd_ref[...] = jnp.dot(h, wa_ref[...], preferred_element_type=jnp.float32)


def _compute_init(frag_node, W_init, We_top, wa_top):
    return pl.pallas_call(
        _init_body,
        grid=(_N // _NBLK,),
        in_specs=[pl.BlockSpec((_NBLK, _D), lambda i: (i, 0)),
                  pl.BlockSpec((_D, _D), lambda i: (0, 0)),
                  pl.BlockSpec((_D, _D), lambda i: (0, 0)),
                  pl.BlockSpec((_D, 1), lambda i: (0, 0))],
        out_specs=[pl.BlockSpec((_NBLK, _D), lambda i: (i, 0)),
                   pl.BlockSpec((_NBLK, _D), lambda i: (i, 0)),
                   pl.BlockSpec((_NBLK, 1), lambda i: (i, 0))],
        out_shape=[jax.ShapeDtypeStruct((_N, _D), jnp.float32),
                   jax.ShapeDtypeStruct((_N, _D), jnp.float32),
                   jax.ShapeDtypeStruct((_N, 1), jnp.float32)],
    )(frag_node, W_init, We_top, wa_top)


def _update_body(u_ref, sig_ref, h_ref, wmsg_ref, wz_ref, uz_ref, wr_ref,
                 ur_ref, wn_ref, un_ref, we_ref, wa_ref,
                 h_out, hn_out, hd_out):
    sig = sig_ref[:, 0] + sig_ref[:, 1]
    U = u_ref[0] + u_ref[1]
    S = U / (sig + 1e-9)[:, None]
    ctx = _elu(jnp.dot(S, wmsg_ref[...], preferred_element_type=jnp.float32))
    h = h_ref[...]
    z = jax.nn.sigmoid(jnp.dot(ctx, wz_ref[...], preferred_element_type=jnp.float32)
                       + jnp.dot(h, uz_ref[...], preferred_element_type=jnp.float32))
    r = jax.nn.sigmoid(jnp.dot(ctx, wr_ref[...], preferred_element_type=jnp.float32)
                       + jnp.dot(h, ur_ref[...], preferred_element_type=jnp.float32))
    n = jnp.tanh(jnp.dot(ctx, wn_ref[...], preferred_element_type=jnp.float32)
                 + r * jnp.dot(h, un_ref[...], preferred_element_type=jnp.float32))
    hn = (1.0 - z) * n + z * h
    h_out[...] = hn
    hn_out[...] = jnp.dot(hn, we_ref[...], preferred_element_type=jnp.float32)
    hd_out[...] = jnp.dot(hn, wa_ref[...], preferred_element_type=jnp.float32)


def _compute_update(U2, sig2, h, W_msg, Wz, Uz, Wr, Ur, Wn, Un, We_top, wa_top):
    wspec = pl.BlockSpec((_D, _D), lambda i: (0, 0))
    return pl.pallas_call(
        _update_body,
        grid=(_N // _NBLK,),
        in_specs=[pl.BlockSpec((_NC, _NBLK, _D), lambda i: (0, i, 0)),
                  pl.BlockSpec((_NBLK, _NC), lambda i: (i, 0)),
                  pl.BlockSpec((_NBLK, _D), lambda i: (i, 0)),
                  wspec, wspec, wspec, wspec, wspec, wspec, wspec, wspec,
                  pl.BlockSpec((_D, 1), lambda i: (0, 0))],
        out_specs=[pl.BlockSpec((_NBLK, _D), lambda i: (i, 0)),
                   pl.BlockSpec((_NBLK, _D), lambda i: (i, 0)),
                   pl.BlockSpec((_NBLK, 1), lambda i: (i, 0))],
        out_shape=[jax.ShapeDtypeStruct((_N, _D), jnp.float32),
                   jax.ShapeDtypeStruct((_N, _D), jnp.float32),
                   jax.ShapeDtypeStruct((_N, 1), jnp.float32)],
    )(U2, sig2, h, W_msg, Wz, Uz, Wr, Ur, Wn, Un, We_top, wa_top)


def _mol_body(h_ref, ids_ref, wmt_ref, wmb_ref, wmsg_ref, wz_ref, uz_ref,
              wr_ref, ur_ref, wn_ref, un_ref, g_out):
    h = h_ref[...]
    ids = ids_ref[...]
    iota_g = lax.broadcasted_iota(jnp.int32, (_G, _N), 0)
    M = (iota_g == ids).astype(jnp.float32)
    iota_n = lax.broadcasted_iota(jnp.int32, (_N, _G), 1)
    MT = (iota_n == ids.reshape(_N, 1)).astype(jnp.float32)
    g = jnp.dot(M, h, preferred_element_type=jnp.float32)
    wmb_row = wmb_ref[...]
    for _ in range(_T):
        gl = jnp.dot(g, wmt_ref[...], preferred_element_type=jnp.float32)
        hl = jnp.sum(h * wmb_row, axis=1, keepdims=True)
        glg = jnp.dot(MT, gl, preferred_element_type=jnp.float32)
        logit = _lrelu(glg + hl)
        ex = jnp.exp(logit)
        sig = jnp.dot(M, ex, preferred_element_type=jnp.float32)
        sigg = jnp.dot(MT, sig, preferred_element_type=jnp.float32)
        w = ex / (sigg + 1e-9)
        U = jnp.dot(M, w * h, preferred_element_type=jnp.float32)
        ctx = _elu(jnp.dot(U, wmsg_ref[...], preferred_element_type=jnp.float32))
        z = jax.nn.sigmoid(jnp.dot(ctx, wz_ref[...], preferred_element_type=jnp.float32)
                           + jnp.dot(g, uz_ref[...], preferred_element_type=jnp.float32))
        r = jax.nn.sigmoid(jnp.dot(ctx, wr_ref[...], preferred_element_type=jnp.float32)
                           + jnp.dot(g, ur_ref[...], preferred_element_type=jnp.float32))
        n = jnp.tanh(jnp.dot(ctx, wn_ref[...], preferred_element_type=jnp.float32)
                     + r * jnp.dot(g, un_ref[...], preferred_element_type=jnp.float32))
        g = (1.0 - z) * n + z * g
    g_out[...] = g


def _compute_mol(h, ids2d, wm_top, wm_bot_row, W_msg_m, Wz, Uz, Wr, Ur, Wn, Un):
    return pl.pallas_call(
        _mol_body,
        in_specs=[pl.BlockSpec((_N, _D), lambda: (0, 0)),
                  pl.BlockSpec((1, _N), lambda: (0, 0)),
                  pl.BlockSpec((_D, 1), lambda: (0, 0)),
                  pl.BlockSpec((1, _D), lambda: (0, 0)),
                  pl.BlockSpec((_D, _D), lambda: (0, 0)),
                  pl.BlockSpec((_D, _D), lambda: (0, 0)),
                  pl.BlockSpec((_D, _D), lambda: (0, 0)),
                  pl.BlockSpec((_D, _D), lambda: (0, 0)),
                  pl.BlockSpec((_D, _D), lambda: (0, 0)),
                  pl.BlockSpec((_D, _D), lambda: (0, 0)),
                  pl.BlockSpec((_D, _D), lambda: (0, 0))],
        out_specs=pl.BlockSpec((_G, _D), lambda: (0, 0)),
        out_shape=jax.ShapeDtypeStruct((_G, _D), jnp.float32),
    )(h, ids2d, wm_top, wm_bot_row, W_msg_m, Wz, Uz, Wr, Ur, Wn, Un)


def kernel(frag_node, frag_edge, edge_index, graph_ids, W_init, W_edge, w_att, W_msg,
           Wz_a, Uz_a, Wr_a, Ur_a, Wn_a, Un_a,
           w_att_m, W_msg_m, Wz_m, Uz_m, Wr_m, Ur_m, Wn_m, Un_m):
    src = edge_index[0]
    dst = edge_index[1]
    We_top = W_edge[:_D]
    wa_top = w_att[:_D]
    wa_bot = w_att[_D:, 0]
    Eb = _compute_eb(frag_edge, W_edge[_D:])
    h, Hn, hd = _compute_init(frag_node, W_init, We_top, wa_top)
    for _ in range(_L):
        U2, sigf = _edge_kernel(src, dst, Hn, hd[:, 0], Eb, wa_bot)
        h, Hn, hd = _compute_update(U2, sigf.reshape(_NC, _N).T, h, W_msg,
                                    Wz_a, Uz_a, Wr_a, Ur_a, Wn_a, Un_a,
                                    We_top, wa_top)
    g = _compute_mol(h, graph_ids.reshape(1, _N), w_att_m[:_D],
                     w_att_m[_D:, 0].reshape(1, _D), W_msg_m,
                     Wz_m, Uz_m, Wr_m, Ur_m, Wn_m, Un_m)
    return g
